# baseline (speedup 1.0000x reference)
import functools

import jax
import jax.numpy as jnp
from jax import lax
from jax.experimental import pallas as pl
from jax.experimental.pallas import tpu as pltpu
from jax.experimental.pallas import tpu_sc as plsc

N = 10000
NP = 10240
E = 320000
D = 128
NC = 2
NS = 16
SLAB = NP // NS

_MESH = plsc.VectorSubcoreMesh(core_axis_name="c", subcore_axis_name="s")

DEG_EPT = E // (NC * NS)
DEG_UNROLL = 5


@functools.partial(
    pl.kernel,
    out_type=jax.ShapeDtypeStruct((NC * NS * NP,), jnp.float32),
    mesh=_MESH,
    scratch_types=[
        pltpu.VMEM((DEG_EPT,), jnp.int32),
        pltpu.VMEM((NP,), jnp.float32),
    ],
    compiler_params=pltpu.CompilerParams(needs_layout_passes=False),
)
def _deg_call(ei_hbm, zero_hbm, out_hbm, idx_v, hist_v):
    c = lax.axis_index("c")
    s = lax.axis_index("s")
    wid = c * NS + s
    pltpu.sync_copy(zero_hbm, hist_v)
    pltpu.sync_copy(ei_hbm.at[pl.ds(E + wid * DEG_EPT, DEG_EPT)], idx_v)

    def body(i, carry):
        base = pl.multiple_of(i * (16 * DEG_UNROLL), 8)
        for j in range(DEG_UNROLL):
            d16 = idx_v[pl.ds(base + j * 16, 16)]
            cnt, last = plsc.scan_count(d16)
            plsc.addupdate_scatter(
                hist_v, [d16], cnt.astype(jnp.float32), mask=last)
        return carry

    lax.fori_loop(0, DEG_EPT // (16 * DEG_UNROLL), body, 0)
    pltpu.sync_copy(hist_v, out_hbm.at[pl.ds(wid * NP, NP)])


AGG_EPT = E // (NC * NS)
AGG_CHUNK = 40
AGG_NCH = AGG_EPT // AGG_CHUNK
NBUF = 5


@functools.partial(
    pl.kernel,
    out_type=jax.ShapeDtypeStruct((NC * NP, D), jnp.float32),
    mesh=_MESH,
    scratch_types=[
        pltpu.VMEM((AGG_EPT,), jnp.int32),
        pltpu.VMEM((AGG_EPT,), jnp.int32),
        pltpu.VMEM((NBUF, AGG_CHUNK, D), jnp.float32),
        pltpu.VMEM_SHARED((NP, D), jnp.float32),
        pltpu.SemaphoreType.DMA((NBUF,)),
        pltpu.SemaphoreType.DMA((NBUF,)),
    ],
)
def _agg_call(g_hbm, zslab_hbm, ei_hbm, out_hbm, srcv, dstv, ring,
              acc_sh, gsem, ssem):
    c = lax.axis_index("c")
    s = lax.axis_index("s")
    r0 = s * SLAB
    wid = c * NS + s

    @pl.when(c == 0)
    def _():
        pltpu.sync_copy(g_hbm.at[pl.ds(r0, SLAB)], acc_sh.at[pl.ds(r0, SLAB)])

    @pl.when(c == 1)
    def _():
        pltpu.sync_copy(zslab_hbm, acc_sh.at[pl.ds(r0, SLAB)])

    e0 = wid * AGG_EPT
    pltpu.sync_copy(ei_hbm.at[pl.ds(e0, AGG_EPT)], srcv)
    pltpu.sync_copy(ei_hbm.at[pl.ds(E + e0, AGG_EPT)], dstv)

    def start_gather(i, b):
        pltpu.async_copy(
            g_hbm.at[srcv.at[pl.ds(i * AGG_CHUNK, AGG_CHUNK)]],
            ring.at[b], gsem.at[b])

    for b in range(NBUF):
        start_gather(b, b)
    plsc.subcore_barrier()

    @pl.loop(0, AGG_NCH, step=NBUF)
    def _(ib):
        for b in range(NBUF):
            i = ib + b
            pltpu.make_async_copy(
                g_hbm.at[srcv.at[pl.ds(0, AGG_CHUNK)]],
                ring.at[b], gsem.at[b]).wait()
            pltpu.async_copy(
                ring.at[b],
                acc_sh.at[dstv.at[pl.ds(i * AGG_CHUNK, AGG_CHUNK)]],
                ssem.at[b], add=True)
            pltpu.make_async_copy(
                ring.at[b], acc_sh.at[dstv.at[pl.ds(0, AGG_CHUNK)]],
                ssem.at[b]).wait()

            @pl.when(i + NBUF < AGG_NCH)
            def _():
                start_gather(i + NBUF, b)

    plsc.subcore_barrier()
    pltpu.sync_copy(acc_sh.at[pl.ds(r0, SLAB)],
                    out_hbm.at[pl.ds(c * NP + r0, SLAB)])


BLK = 2000
NBLK = N // BLK


def _tc0_body(x_ref, w_ref, h_ref):
    h_ref[...] = jnp.dot(x_ref[...], w_ref[...],
                         preferred_element_type=jnp.float32)


def _tc1_body(h_ref, degp_ref, g_ref, dis_ref):
    deg = jnp.sum(degp_ref[...], axis=0)[:N, None] + 1.0
    dis = lax.rsqrt(deg)
    dis_ref[...] = dis
    g_ref[0:N] = h_ref[...] * dis
    g_ref[N:NP] = jnp.zeros((NP - N, D), jnp.float32)


def _tc2_body(s_ref, dis_ref, b_ref, w_ref, res_ref, g_ref):
    dis = dis_ref[...]
    sfull = s_ref[0] + s_ref[1]
    h = jnp.maximum(sfull * dis + b_ref[...], 0.0)
    res_ref[...] = h
    g = jnp.dot(h, w_ref[...], preferred_element_type=jnp.float32)
    g_ref[...] = g * dis


def _tc3_body(s_ref, dis_ref, b_ref, out_ref):
    sfull = s_ref[0] + s_ref[1]
    out_ref[...] = sfull * dis_ref[...] + b_ref[...]


_tc0 = pl.pallas_call(
    _tc0_body,
    grid=(NBLK,),
    in_specs=[pl.BlockSpec((BLK, D), lambda i: (i, 0)),
              pl.BlockSpec((D, D), lambda i: (0, 0))],
    out_specs=pl.BlockSpec((BLK, D), lambda i: (i, 0)),
    out_shape=jax.ShapeDtypeStruct((N, D), jnp.float32),
)

_tc1 = pl.pallas_call(
    _tc1_body,
    out_shape=(jax.ShapeDtypeStruct((NP, D), jnp.float32),
               jax.ShapeDtypeStruct((N, 1), jnp.float32)),
)

_tc2 = pl.pallas_call(
    _tc2_body,
    grid=(NBLK,),
    in_specs=[pl.BlockSpec((NC, BLK, D), lambda i: (0, i, 0)),
              pl.BlockSpec((BLK, 1), lambda i: (i, 0)),
              pl.BlockSpec((1, D), lambda i: (0, 0)),
              pl.BlockSpec((D, D), lambda i: (0, 0))],
    out_specs=(pl.BlockSpec((BLK, D), lambda i: (i, 0)),
               pl.BlockSpec((BLK, D), lambda i: (i, 0))),
    out_shape=(jax.ShapeDtypeStruct((N, D), jnp.float32),
               jax.ShapeDtypeStruct((NP, D), jnp.float32)),
)

_tc3 = pl.pallas_call(
    _tc3_body,
    grid=(NBLK,),
    in_specs=[pl.BlockSpec((NC, BLK, D), lambda i: (0, i, 0)),
              pl.BlockSpec((BLK, 1), lambda i: (i, 0)),
              pl.BlockSpec((1, D), lambda i: (0, 0))],
    out_specs=pl.BlockSpec((BLK, D), lambda i: (i, 0)),
    out_shape=jax.ShapeDtypeStruct((N, D), jnp.float32),
)


def kernel(x, edge_index, percent, ricci_curvature, W1, b1, W2, b2):
    del percent, ricci_curvature
    ei = edge_index.reshape(2 * E)
    zero_np = jnp.zeros((NP,), jnp.float32)
    zslab = jnp.zeros((SLAB, D), jnp.float32)

    degp = _deg_call(ei, zero_np).reshape(NC * NS, NP)
    h1 = _tc0(x, W1)
    g1, dis = _tc1(h1, degp)
    s1 = _agg_call(g1, zslab, ei).reshape(NC, NP, D)
    res, g2 = _tc2(s1, dis, b1.reshape(1, D), W2)
    s2 = _agg_call(g2, zslab, ei).reshape(NC, NP, D)
    out = _tc3(s2, dis, b2.reshape(1, D))
    return (out, res)


# speedup vs baseline: 1.0039x; 1.0039x over previous
import functools

import jax
import jax.numpy as jnp
from jax import lax
from jax.experimental import pallas as pl
from jax.experimental.pallas import tpu as pltpu
from jax.experimental.pallas import tpu_sc as plsc

N = 10000
NP = 10240
E = 320000
D = 128
NC = 2
NS = 16
SLAB = NP // NS

_MESH = plsc.VectorSubcoreMesh(core_axis_name="c", subcore_axis_name="s")

DEG_EPT = E // (NC * NS)
DEG_UNROLL = 5


@functools.partial(
    pl.kernel,
    out_type=jax.ShapeDtypeStruct((NC * NS * NP,), jnp.float32),
    mesh=_MESH,
    scratch_types=[
        pltpu.VMEM((DEG_EPT,), jnp.int32),
        pltpu.VMEM((NP,), jnp.float32),
    ],
    compiler_params=pltpu.CompilerParams(needs_layout_passes=False),
)
def _deg_call(ei_hbm, zero_hbm, out_hbm, idx_v, hist_v):
    c = lax.axis_index("c")
    s = lax.axis_index("s")
    wid = c * NS + s
    pltpu.sync_copy(zero_hbm, hist_v)
    pltpu.sync_copy(ei_hbm.at[pl.ds(E + wid * DEG_EPT, DEG_EPT)], idx_v)

    def body(i, carry):
        base = pl.multiple_of(i * (16 * DEG_UNROLL), 8)
        for j in range(DEG_UNROLL):
            d16 = idx_v[pl.ds(base + j * 16, 16)]
            cnt, last = plsc.scan_count(d16)
            plsc.addupdate_scatter(
                hist_v, [d16], cnt.astype(jnp.float32), mask=last)
        return carry

    lax.fori_loop(0, DEG_EPT // (16 * DEG_UNROLL), body, 0)
    pltpu.sync_copy(hist_v, out_hbm.at[pl.ds(wid * NP, NP)])


AGG_EPT = E // (NC * NS)
AGG_CHUNK = 40
AGG_NCH = AGG_EPT // AGG_CHUNK
NBUF = 5


@functools.partial(
    pl.kernel,
    out_type=jax.ShapeDtypeStruct((NC * NP, D), jnp.float32),
    mesh=_MESH,
    scratch_types=[
        pltpu.VMEM((AGG_EPT,), jnp.int32),
        pltpu.VMEM((AGG_EPT,), jnp.int32),
        pltpu.VMEM((NBUF, AGG_CHUNK, D), jnp.float32),
        pltpu.VMEM_SHARED((NP, D), jnp.float32),
        pltpu.SemaphoreType.DMA((NBUF,)),
        pltpu.SemaphoreType.DMA((NBUF,)),
    ],
)
def _agg_call(g_hbm, zslab_hbm, ei_hbm, out_hbm, srcv, dstv, ring,
              acc_sh, gsem, ssem):
    c = lax.axis_index("c")
    s = lax.axis_index("s")
    r0 = s * SLAB
    wid = c * NS + s

    @pl.when(c == 0)
    def _():
        pltpu.sync_copy(g_hbm.at[pl.ds(r0, SLAB)], acc_sh.at[pl.ds(r0, SLAB)])

    @pl.when(c == 1)
    def _():
        pltpu.sync_copy(zslab_hbm, acc_sh.at[pl.ds(r0, SLAB)])

    e0 = wid * AGG_EPT
    pltpu.sync_copy(ei_hbm.at[pl.ds(e0, AGG_EPT)], srcv)
    pltpu.sync_copy(ei_hbm.at[pl.ds(E + e0, AGG_EPT)], dstv)

    def start_gather(i, b):
        pltpu.async_copy(
            g_hbm.at[srcv.at[pl.ds(i * AGG_CHUNK, AGG_CHUNK)]],
            ring.at[b], gsem.at[b])

    for b in range(NBUF):
        start_gather(b, b)
    plsc.subcore_barrier()

    @pl.loop(0, AGG_NCH, step=NBUF)
    def _(ib):
        for b in range(NBUF):
            i = ib + b
            pltpu.make_async_copy(
                g_hbm.at[srcv.at[pl.ds(0, AGG_CHUNK)]],
                ring.at[b], gsem.at[b]).wait()
            pltpu.async_copy(
                ring.at[b],
                acc_sh.at[dstv.at[pl.ds(i * AGG_CHUNK, AGG_CHUNK)]],
                ssem.at[b], add=True)
            pltpu.make_async_copy(
                ring.at[b], acc_sh.at[dstv.at[pl.ds(0, AGG_CHUNK)]],
                ssem.at[b]).wait()

            @pl.when(i + NBUF < AGG_NCH)
            def _():
                start_gather(i + NBUF, b)

    plsc.subcore_barrier()
    pltpu.sync_copy(acc_sh.at[pl.ds(r0, SLAB)],
                    out_hbm.at[pl.ds(c * NP + r0, SLAB)])


def _tc0_body(x_ref, w_ref, h_ref):
    h_ref[...] = jnp.dot(x_ref[...], w_ref[...],
                         preferred_element_type=jnp.float32)


def _tc1_body(h_ref, degp_ref, g_ref, dis_ref):
    deg = jnp.sum(degp_ref[...], axis=0)[:N, None] + 1.0
    dis = lax.rsqrt(deg)
    dis_ref[...] = dis
    g_ref[0:N] = h_ref[...] * dis
    g_ref[N:NP] = jnp.zeros((NP - N, D), jnp.float32)


def _tc2_body(s_ref, dis_ref, b_ref, w_ref, res_ref, g_ref):
    dis = dis_ref[...]
    sfull = s_ref[0, 0:N] + s_ref[1, 0:N]
    h = jnp.maximum(sfull * dis + b_ref[...], 0.0)
    res_ref[...] = h
    g = jnp.dot(h, w_ref[...], preferred_element_type=jnp.float32)
    g_ref[0:N] = g * dis
    g_ref[N:NP] = jnp.zeros((NP - N, D), jnp.float32)


def _tc3_body(s_ref, dis_ref, b_ref, out_ref):
    sfull = s_ref[0, 0:N] + s_ref[1, 0:N]
    out_ref[...] = sfull * dis_ref[...] + b_ref[...]


_tc0 = pl.pallas_call(
    _tc0_body,
    out_shape=jax.ShapeDtypeStruct((N, D), jnp.float32),
)

_tc1 = pl.pallas_call(
    _tc1_body,
    out_shape=(jax.ShapeDtypeStruct((NP, D), jnp.float32),
               jax.ShapeDtypeStruct((N, 1), jnp.float32)),
)

_tc2 = pl.pallas_call(
    _tc2_body,
    out_shape=(jax.ShapeDtypeStruct((N, D), jnp.float32),
               jax.ShapeDtypeStruct((NP, D), jnp.float32)),
)

_tc3 = pl.pallas_call(
    _tc3_body,
    out_shape=jax.ShapeDtypeStruct((N, D), jnp.float32),
)


def kernel(x, edge_index, percent, ricci_curvature, W1, b1, W2, b2):
    del percent, ricci_curvature
    ei = edge_index.reshape(2 * E)
    zero_np = jnp.zeros((NP,), jnp.float32)
    zslab = jnp.zeros((SLAB, D), jnp.float32)

    degp = _deg_call(ei, zero_np).reshape(NC * NS, NP)
    h1 = _tc0(x, W1)
    g1, dis = _tc1(h1, degp)
    s1 = _agg_call(g1, zslab, ei).reshape(NC, NP, D)
    res, g2 = _tc2(s1, dis, b1.reshape(1, D), W2)
    s2 = _agg_call(g2, zslab, ei).reshape(NC, NP, D)
    out = _tc3(s2, dis, b2.reshape(1, D))
    return (out, res)
